# baseline (speedup 1.0000x reference)
import jax
import jax.numpy as jnp
from jax import lax
from jax.experimental import pallas as pl
from jax.experimental.pallas import tpu as pltpu
from jax.experimental.pallas import tpu_sc as plsc

VOCAB = 6
DIM = 4
BATCH = 4096
HIST = 200

NC = 2
NS = 16
L = 16
NW = NC * NS

BPW = BATCH // NW
LC = 8
NCH = HIST // LC
BV = BPW // L


def _sc_body(e_hbm, h_hbm, idx_hbm, out_hbm,
             e_v, h_v, tab_v, tok_v, out0, out1, osem):
    wid = lax.axis_index("s") * NC + lax.axis_index("c")

    pltpu.sync_copy(e_hbm, e_v)
    pltpu.sync_copy(h_hbm, h_v)
    iota = lax.iota(jnp.int32, L)
    for t in range(4):
        i = iota + (16 * t)
        v = i >> 3
        c = i & 7
        acc = jnp.zeros((L,), jnp.float32)
        for d in range(DIM):
            ev = plsc.load_gather(e_v, [v * DIM + d])
            hv = plsc.load_gather(h_v, [c * DIM + d])
            acc = acc + ev * hv
        tab_v[pl.ds(16 * t, 16)] = acc

    base_b = wid * BPW
    pltpu.sync_copy(idx_hbm.at[:, pl.ds(base_b, BPW)], tok_v)

    bpat = [iota + 16 * t for t in range(BV)]
    out_bufs = [out0, out1]

    def out_copy(ch, buf):
        return pltpu.make_async_copy(
            buf,
            out_hbm.at[:, pl.ds(ch * LC, LC), pl.ds(base_b, BPW)],
            osem,
        )

    def compute_chunk(ch, out_v):
        l0 = ch * LC
        for dl in range(LC):
            lvec = jnp.zeros((L,), jnp.int32) + (l0 + dl)
            toks = [plsc.load_gather(tok_v, [lvec, bpat[t]]) for t in range(BV)]
            bases = [tok << 3 for tok in toks]
            for c in range(VOCAB):
                vals = [plsc.load_gather(tab_v, [bases[t] + c]) for t in range(BV)]
                for t in range(BV):
                    out_v[c, dl, pl.ds(16 * t, 16)] = vals[t]

    def chunk_pair(i, _):
        for s in range(2):
            ch = i * 2 + s
            out_v = out_bufs[s]

            @pl.when(i > 0)
            def _():
                out_copy(ch - 2, out_v).wait()

            compute_chunk(ch, out_v)
            out_copy(ch, out_v).start()
        return 0

    lax.fori_loop(0, (NCH - 1) // 2, chunk_pair, 0)
    out_copy(NCH - 3, out0).wait()
    compute_chunk(NCH - 1, out0)
    out_copy(NCH - 1, out0).start()
    out_copy(NCH - 2, out1).wait()
    out_copy(NCH - 1, out0).wait()


def kernel(inputs, embed_weight, lm_head_weight):
    idx = inputs.astype(jnp.int32).T
    e_pad = jnp.pad(embed_weight.reshape(VOCAB * DIM), (0, 32 - VOCAB * DIM))
    h_pad = jnp.pad(lm_head_weight.reshape(VOCAB * DIM), (0, 32 - VOCAB * DIM))

    mesh = plsc.VectorSubcoreMesh(
        core_axis_name="c", subcore_axis_name="s", num_cores=NC, num_subcores=NS
    )
    run = pl.kernel(
        _sc_body,
        out_type=jax.ShapeDtypeStruct((VOCAB, HIST, BATCH), jnp.float32),
        mesh=mesh,
        compiler_params=pltpu.CompilerParams(
            needs_layout_passes=False, use_tc_tiling_on_sc=True
        ),
        scratch_types=[
            pltpu.VMEM((32,), jnp.float32),
            pltpu.VMEM((32,), jnp.float32),
            pltpu.VMEM((64,), jnp.float32),
            pltpu.VMEM((HIST, BPW), jnp.int32),
            pltpu.VMEM((VOCAB, LC, BPW), jnp.float32),
            pltpu.VMEM((VOCAB, LC, BPW), jnp.float32),
            pltpu.SemaphoreType.DMA,
        ],
    )
    out_t = run(e_pad, h_pad, idx)
    return out_t.transpose(2, 1, 0)


# speedup vs baseline: 1.0115x; 1.0115x over previous
import jax
import jax.numpy as jnp
from jax import lax
from jax.experimental import pallas as pl
from jax.experimental.pallas import tpu as pltpu
from jax.experimental.pallas import tpu_sc as plsc

VOCAB = 6
DIM = 4
BATCH = 4096
HIST = 200

NC = 2
NS = 16
L = 16
NW = NC * NS

BPW = BATCH // NW
LC = 8
NCH = HIST // LC
BV = BPW // L


def _sc_body(e_hbm, h_hbm, idx_hbm, out_hbm,
             e_v, h_v, tab_v, tok_v, out0, out1, osem):
    wid = lax.axis_index("s") * NC + lax.axis_index("c")

    pltpu.sync_copy(e_hbm, e_v)
    pltpu.sync_copy(h_hbm, h_v)
    iota = lax.iota(jnp.int32, L)
    for t in range(4):
        i = iota + (16 * t)
        v = i >> 3
        c = i & 7
        acc = jnp.zeros((L,), jnp.float32)
        for d in range(DIM):
            ev = plsc.load_gather(e_v, [v * DIM + d])
            hv = plsc.load_gather(h_v, [c * DIM + d])
            acc = acc + ev * hv
        tab_v[pl.ds(16 * t, 16)] = acc

    base_b = wid * BPW
    pltpu.sync_copy(idx_hbm.at[:, pl.ds(base_b, BPW)], tok_v)

    bpat = [iota + 16 * t for t in range(BV)]
    out_bufs = [out0, out1]

    def out_copy(ch, buf):
        return pltpu.make_async_copy(
            buf,
            out_hbm.at[:, pl.ds(ch * LC, LC), pl.ds(base_b, BPW)],
            osem,
        )

    def compute_chunk(ch, out_v):
        l0 = ch * LC

        def row_loads(dl):
            lvec = jnp.zeros((L,), jnp.int32) + (l0 + dl)
            toks = [plsc.load_gather(tok_v, [lvec, bpat[t]]) for t in range(BV)]
            bases = [tok << 3 for tok in toks]
            return [
                [plsc.load_gather(tab_v, [bases[t] + c]) for t in range(BV)]
                for c in range(VOCAB)
            ]

        vals = row_loads(0)
        for dl in range(LC):
            nxt = row_loads(dl + 1) if dl + 1 < LC else None
            for c in range(VOCAB):
                for t in range(BV):
                    out_v[c, dl, pl.ds(16 * t, 16)] = vals[c][t]
            vals = nxt

    def chunk_pair(i, _):
        for s in range(2):
            ch = i * 2 + s
            out_v = out_bufs[s]

            @pl.when(i > 0)
            def _():
                out_copy(ch - 2, out_v).wait()

            compute_chunk(ch, out_v)
            out_copy(ch, out_v).start()
        return 0

    lax.fori_loop(0, (NCH - 1) // 2, chunk_pair, 0)
    out_copy(NCH - 3, out0).wait()
    compute_chunk(NCH - 1, out0)
    out_copy(NCH - 1, out0).start()
    out_copy(NCH - 2, out1).wait()
    out_copy(NCH - 1, out0).wait()


def kernel(inputs, embed_weight, lm_head_weight):
    idx = inputs.astype(jnp.int32).T
    e_pad = jnp.pad(embed_weight.reshape(VOCAB * DIM), (0, 32 - VOCAB * DIM))
    h_pad = jnp.pad(lm_head_weight.reshape(VOCAB * DIM), (0, 32 - VOCAB * DIM))

    mesh = plsc.VectorSubcoreMesh(
        core_axis_name="c", subcore_axis_name="s", num_cores=NC, num_subcores=NS
    )
    run = pl.kernel(
        _sc_body,
        out_type=jax.ShapeDtypeStruct((VOCAB, HIST, BATCH), jnp.float32),
        mesh=mesh,
        compiler_params=pltpu.CompilerParams(
            needs_layout_passes=False, use_tc_tiling_on_sc=True
        ),
        scratch_types=[
            pltpu.VMEM((32,), jnp.float32),
            pltpu.VMEM((32,), jnp.float32),
            pltpu.VMEM((64,), jnp.float32),
            pltpu.VMEM((HIST, BPW), jnp.int32),
            pltpu.VMEM((VOCAB, LC, BPW), jnp.float32),
            pltpu.VMEM((VOCAB, LC, BPW), jnp.float32),
            pltpu.SemaphoreType.DMA,
        ],
    )
    out_t = run(e_pad, h_pad, idx)
    return out_t.transpose(2, 1, 0)
